# baseline (speedup 1.0000x reference)
import functools

import jax
import jax.numpy as jnp
from jax import lax
from jax.experimental import pallas as pl
from jax.experimental.pallas import tpu as pltpu
from jax.experimental.pallas import tpu_sc as plsc

B, L, V, D, P, T = 4096, 200, 100000, 64, 512, 2
EPS = 1e-12

NC, NS = 2, 16
NW = NC * NS
N = B * L
ROWS_PER_W = N // NW
CHUNK = 128
NCHUNKS = ROWS_PER_W // CHUNK
NGROUPS = CHUNK // 16


def _ehr_body(cid_hbm, cmb_hbm, code_hbm, comb_hbm, gam_hbm, bet_hbm,
              out_hbm,
              cid_v0, cid_v1, cmb_v0, cmb_v1, rows_v0, rows_v1,
              out_v0, out_v1, comb_v, gam_v, bet_v,
              ids_s0, ids_s1, g_s0, g_s1, o_s0, o_s1):
    wid = lax.axis_index("s") * NC + lax.axis_index("c")
    cbase = wid * NCHUNKS

    cid_v = (cid_v0, cid_v1)
    cmb_v = (cmb_v0, cmb_v1)
    rows_v = (rows_v0, rows_v1)
    out_v = (out_v0, out_v1)
    ids_s = (ids_s0, ids_s1)
    g_s = (g_s0, g_s1)
    o_s = (o_s0, o_s1)

    pltpu.sync_copy(comb_hbm, comb_v)
    pltpu.sync_copy(gam_hbm, gam_v)
    pltpu.sync_copy(bet_hbm, bet_v)

    inv_d = jnp.float32(1.0 / D)

    def cid_copy(c, p):
        return pltpu.make_async_copy(cid_hbm.at[cbase + c], cid_v[p],
                                     ids_s[p])

    def cmb_copy(c, p):
        return pltpu.make_async_copy(cmb_hbm.at[cbase + c], cmb_v[p],
                                     ids_s[p])

    def ids_start(c, p):
        cid_copy(c, p).start()
        cmb_copy(c, p).start()

    def ids_wait(c, p):
        cid_copy(c, p).wait()
        cmb_copy(c, p).wait()

    def gather_copy(c, p):
        return pltpu.make_async_copy(code_hbm.at[cid_v[p]], rows_v[p], g_s[p])

    def out_copy(c, p):
        off = (cbase + c) * CHUNK
        return pltpu.make_async_copy(out_v[p], out_hbm.at[pl.ds(off, CHUNK)],
                                     o_s[p])

    gam = [gam_v[pl.ds(k * 16, 16)] for k in range(4)]
    bet = [bet_v[pl.ds(k * 16, 16)] for k in range(4)]
    iota16 = lax.iota(jnp.int32, 16)
    idx15 = jnp.full((16,), 15, jnp.int32)
    lane_idx = [jnp.full((16,), j, jnp.int32) for j in range(16)]

    def compute(c, p):
        rv, ov, iv = rows_v[p], out_v[p], cmb_v[p]

        @pl.loop(0, NGROUPS)
        def _group(g):
            cvec = iv[pl.ds(g * 16, 16)]
            s_l = jnp.zeros((16,), jnp.float32)
            q_l = jnp.zeros((16,), jnp.float32)
            for j in range(16):
                r = g * 16 + j
                base = jnp.take_along_axis(cvec, lane_idx[j], axis=0) * 4
                v = [rv[r, pl.ds(k * 16, 16)]
                     + plsc.load_gather(comb_v, [base + k, iota16])
                     for k in range(4)]
                for k in range(4):
                    ov[r, pl.ds(k * 16, 16)] = v[k]
                t = (v[0] + v[1]) + (v[2] + v[3])
                q = (v[0] * v[0] + v[1] * v[1]) + (v[2] * v[2] + v[3] * v[3])
                ts = jnp.take_along_axis(plsc.cumsum(t), idx15, axis=0)
                qs = jnp.take_along_axis(plsc.cumsum(q), idx15, axis=0)
                mask = iota16 == j
                s_l = jnp.where(mask, ts, s_l)
                q_l = jnp.where(mask, qs, q_l)
            mu_l = s_l * inv_d
            var_l = q_l * inv_d - mu_l * mu_l
            x = var_l + EPS
            i = plsc.bitcast(x, jnp.int32)
            i = jnp.int32(0x5F3759DF) - lax.shift_right_logical(i, 1)
            y = plsc.bitcast(i, jnp.float32)
            y = y * (1.5 - 0.5 * x * y * y)
            y = y * (1.5 - 0.5 * x * y * y)
            y = y * (1.5 - 0.5 * x * y * y)
            rstd_l = y
            for j in range(16):
                r = g * 16 + j
                muj = jnp.take_along_axis(mu_l, lane_idx[j], axis=0)
                rsj = jnp.take_along_axis(rstd_l, lane_idx[j], axis=0)
                for k in range(4):
                    vk = ov[r, pl.ds(k * 16, 16)]
                    ov[r, pl.ds(k * 16, 16)] = ((vk - muj) * rsj * gam[k]
                                                + bet[k])

    ids_start(0, 0)
    ids_start(1, 1)
    ids_wait(0, 0)
    gather_copy(0, 0).start()

    @pl.loop(0, NCHUNKS // 2)
    def _chunk2(c2):
        for p in (0, 1):
            c = c2 * 2 + p
            q = 1 - p

            @pl.when(c + 1 < NCHUNKS)
            def _():
                ids_wait(c + 1, q)
                gather_copy(c + 1, q).start()

            gather_copy(c, p).wait()

            @pl.when(c >= 2)
            def _():
                out_copy(c - 2, p).wait()

            compute(c, p)

            @pl.when(c + 2 < NCHUNKS)
            def _():
                ids_start(c + 2, p)

            out_copy(c, p).start()

    out_copy(NCHUNKS - 2, 0).wait()
    out_copy(NCHUNKS - 1, 1).wait()


@jax.jit
def _ehr(cid3, cmb3, code_table, comb_table, gam, bet):
    mesh = plsc.VectorSubcoreMesh(core_axis_name="c", subcore_axis_name="s")
    return pl.kernel(
        _ehr_body,
        out_type=jax.ShapeDtypeStruct((N, D), jnp.float32),
        mesh=mesh,
        compiler_params=pltpu.CompilerParams(
            needs_layout_passes=False, use_tc_tiling_on_sc=False),
        scratch_types=[
            pltpu.VMEM((CHUNK,), jnp.int32),
            pltpu.VMEM((CHUNK,), jnp.int32),
            pltpu.VMEM((CHUNK,), jnp.int32),
            pltpu.VMEM((CHUNK,), jnp.int32),
            pltpu.VMEM((CHUNK, D), jnp.float32),
            pltpu.VMEM((CHUNK, D), jnp.float32),
            pltpu.VMEM((CHUNK, D), jnp.float32),
            pltpu.VMEM((CHUNK, D), jnp.float32),
            pltpu.VMEM((P * T * 4, 16), jnp.float32),
            pltpu.VMEM((D,), jnp.float32),
            pltpu.VMEM((D,), jnp.float32),
            pltpu.SemaphoreType.DMA,
            pltpu.SemaphoreType.DMA,
            pltpu.SemaphoreType.DMA,
            pltpu.SemaphoreType.DMA,
            pltpu.SemaphoreType.DMA,
            pltpu.SemaphoreType.DMA,
        ],
    )(cid3, cmb3, code_table, comb_table, gam, bet)


def kernel(code_ids, position_ids, segment_ids, code_table, pos_table,
           seg_table, ln_gamma, ln_beta):
    nch = N // CHUNK
    cid3 = code_ids.reshape(nch, CHUNK).astype(jnp.int32)
    cmb3 = (position_ids.astype(jnp.int32) * T
            + segment_ids.astype(jnp.int32)).reshape(nch, CHUNK)
    comb_table = (pos_table[:, None, :] + seg_table[None, :, :]).reshape(
        P * T * 4, 16)
    out = _ehr(cid3, cmb3, code_table, comb_table, ln_gamma, ln_beta)
    return out.reshape(code_ids.shape + (D,))


# speedup vs baseline: 1.0708x; 1.0708x over previous
import functools

import jax
import jax.numpy as jnp
from jax import lax
from jax.experimental import pallas as pl
from jax.experimental.pallas import tpu as pltpu
from jax.experimental.pallas import tpu_sc as plsc

B, L, V, D, P, T = 4096, 200, 100000, 64, 512, 2
EPS = 1e-12

NC, NS = 2, 16
NW = NC * NS
N = B * L
ROWS_PER_W = N // NW
CHUNK = 128
NCHUNKS = ROWS_PER_W // CHUNK
NGROUPS = CHUNK // 16


def _ehr_body(cid_hbm, cmb_hbm, code_hbm, comb_hbm, gam_hbm, bet_hbm,
              out_hbm,
              cid_v0, cid_v1, cmb_v0, cmb_v1, rows_v0, rows_v1,
              out_v0, out_v1, comb_v, gam_v, bet_v,
              ids_s0, ids_s1, g_s0, g_s1, o_s0, o_s1):
    wid = lax.axis_index("s") * NC + lax.axis_index("c")
    cbase = wid * NCHUNKS

    cid_v = (cid_v0, cid_v1)
    cmb_v = (cmb_v0, cmb_v1)
    rows_v = (rows_v0, rows_v1)
    out_v = (out_v0, out_v1)
    ids_s = (ids_s0, ids_s1)
    g_s = (g_s0, g_s1)
    o_s = (o_s0, o_s1)

    pltpu.sync_copy(comb_hbm, comb_v)
    pltpu.sync_copy(gam_hbm, gam_v)
    pltpu.sync_copy(bet_hbm, bet_v)

    inv_d = jnp.float32(1.0 / D)

    def cid_copy(c, p):
        return pltpu.make_async_copy(cid_hbm.at[cbase + c], cid_v[p],
                                     ids_s[p])

    def cmb_copy(c, p):
        return pltpu.make_async_copy(cmb_hbm.at[cbase + c], cmb_v[p],
                                     ids_s[p])

    def ids_start(c, p):
        cid_copy(c, p).start()
        cmb_copy(c, p).start()

    def ids_wait(c, p):
        cid_copy(c, p).wait()
        cmb_copy(c, p).wait()

    def gather_copy(c, p):
        return pltpu.make_async_copy(code_hbm.at[cid_v[p]], rows_v[p], g_s[p])

    def out_copy(c, p):
        off = (cbase + c) * CHUNK
        return pltpu.make_async_copy(out_v[p], out_hbm.at[pl.ds(off, CHUNK)],
                                     o_s[p])

    gam = [gam_v[pl.ds(k * 16, 16)] for k in range(4)]
    bet = [bet_v[pl.ds(k * 16, 16)] for k in range(4)]
    iota16 = lax.iota(jnp.int32, 16)
    idx15 = jnp.full((16,), 15, jnp.int32)
    lane_idx = [jnp.full((16,), j, jnp.int32) for j in range(16)]

    def compute(c, p):
        rv, ov, iv = rows_v[p], out_v[p], cmb_v[p]

        @pl.loop(0, NGROUPS)
        def _group(g):
            cvec = iv[pl.ds(g * 16, 16)]
            s_l = jnp.zeros((16,), jnp.float32)
            q_l = jnp.zeros((16,), jnp.float32)
            for j in range(16):
                r = g * 16 + j
                cm = cvec[j]
                v = [rv[r, pl.ds(k * 16, 16)] + comb_v[cm, pl.ds(k * 16, 16)]
                     for k in range(4)]
                for k in range(4):
                    ov[r, pl.ds(k * 16, 16)] = v[k]
                t = (v[0] + v[1]) + (v[2] + v[3])
                q = (v[0] * v[0] + v[1] * v[1]) + (v[2] * v[2] + v[3] * v[3])
                ts = jnp.take_along_axis(plsc.cumsum(t), idx15, axis=0)
                qs = jnp.take_along_axis(plsc.cumsum(q), idx15, axis=0)
                mask = iota16 == j
                s_l = jnp.where(mask, ts, s_l)
                q_l = jnp.where(mask, qs, q_l)
            mu_l = s_l * inv_d
            var_l = q_l * inv_d - mu_l * mu_l
            x = var_l + EPS
            i = plsc.bitcast(x, jnp.int32)
            i = jnp.int32(0x5F3759DF) - lax.shift_right_logical(i, 1)
            y = plsc.bitcast(i, jnp.float32)
            y = y * (1.5 - 0.5 * x * y * y)
            y = y * (1.5 - 0.5 * x * y * y)
            y = y * (1.5 - 0.5 * x * y * y)
            rstd_l = y
            for j in range(16):
                r = g * 16 + j
                muj = jnp.take_along_axis(mu_l, lane_idx[j], axis=0)
                rsj = jnp.take_along_axis(rstd_l, lane_idx[j], axis=0)
                for k in range(4):
                    vk = ov[r, pl.ds(k * 16, 16)]
                    ov[r, pl.ds(k * 16, 16)] = ((vk - muj) * rsj * gam[k]
                                                + bet[k])

    ids_start(0, 0)
    ids_start(1, 1)
    ids_wait(0, 0)
    gather_copy(0, 0).start()

    @pl.loop(0, NCHUNKS // 2)
    def _chunk2(c2):
        for p in (0, 1):
            c = c2 * 2 + p
            q = 1 - p

            @pl.when(c + 1 < NCHUNKS)
            def _():
                ids_wait(c + 1, q)
                gather_copy(c + 1, q).start()

            gather_copy(c, p).wait()

            @pl.when(c >= 2)
            def _():
                out_copy(c - 2, p).wait()

            compute(c, p)

            @pl.when(c + 2 < NCHUNKS)
            def _():
                ids_start(c + 2, p)

            out_copy(c, p).start()

    out_copy(NCHUNKS - 2, 0).wait()
    out_copy(NCHUNKS - 1, 1).wait()


@jax.jit
def _ehr(cid3, cmb3, code_table, comb_table, gam, bet):
    mesh = plsc.VectorSubcoreMesh(core_axis_name="c", subcore_axis_name="s")
    return pl.kernel(
        _ehr_body,
        out_type=jax.ShapeDtypeStruct((N, D), jnp.float32),
        mesh=mesh,
        compiler_params=pltpu.CompilerParams(
            needs_layout_passes=False, use_tc_tiling_on_sc=False),
        scratch_types=[
            pltpu.VMEM((CHUNK,), jnp.int32),
            pltpu.VMEM((CHUNK,), jnp.int32),
            pltpu.VMEM((CHUNK,), jnp.int32),
            pltpu.VMEM((CHUNK,), jnp.int32),
            pltpu.VMEM((CHUNK, D), jnp.float32),
            pltpu.VMEM((CHUNK, D), jnp.float32),
            pltpu.VMEM((CHUNK, D), jnp.float32),
            pltpu.VMEM((CHUNK, D), jnp.float32),
            pltpu.VMEM((P * T, D), jnp.float32),
            pltpu.VMEM((D,), jnp.float32),
            pltpu.VMEM((D,), jnp.float32),
            pltpu.SemaphoreType.DMA,
            pltpu.SemaphoreType.DMA,
            pltpu.SemaphoreType.DMA,
            pltpu.SemaphoreType.DMA,
            pltpu.SemaphoreType.DMA,
            pltpu.SemaphoreType.DMA,
        ],
    )(cid3, cmb3, code_table, comb_table, gam, bet)


def kernel(code_ids, position_ids, segment_ids, code_table, pos_table,
           seg_table, ln_gamma, ln_beta):
    nch = N // CHUNK
    cid3 = code_ids.reshape(nch, CHUNK).astype(jnp.int32)
    cmb3 = (position_ids.astype(jnp.int32) * T
            + segment_ids.astype(jnp.int32)).reshape(nch, CHUNK)
    comb_table = (pos_table[:, None, :] + seg_table[None, :, :]).reshape(
        P * T, D)
    out = _ehr(cid3, cmb3, code_table, comb_table, ln_gamma, ln_beta)
    return out.reshape(code_ids.shape + (D,))


# speedup vs baseline: 1.1070x; 1.0338x over previous
import functools

import jax
import jax.numpy as jnp
from jax import lax
from jax.experimental import pallas as pl
from jax.experimental.pallas import tpu as pltpu
from jax.experimental.pallas import tpu_sc as plsc

B, L, V, D, P, T = 4096, 200, 100000, 64, 512, 2
EPS = 1e-12

NC, NS = 2, 16
NW = NC * NS
N = B * L
ROWS_PER_W = N // NW
CHUNK = 128
NCHUNKS = ROWS_PER_W // CHUNK
NGROUPS = CHUNK // 16


def _ehr_body(cid_hbm, cmb_hbm, code_hbm, comb_hbm, gam_hbm, bet_hbm,
              out_hbm,
              cid_v0, cid_v1, cmb_v0, cmb_v1, rows_v0, rows_v1,
              out_v0, out_v1, comb_v, gam_v, bet_v,
              ids_s0, ids_s1, g_s0, g_s1, o_s0, o_s1):
    wid = lax.axis_index("s") * NC + lax.axis_index("c")
    cbase = wid * NCHUNKS

    cid_v = (cid_v0, cid_v1)
    cmb_v = (cmb_v0, cmb_v1)
    rows_v = (rows_v0, rows_v1)
    out_v = (out_v0, out_v1)
    ids_s = (ids_s0, ids_s1)
    g_s = (g_s0, g_s1)
    o_s = (o_s0, o_s1)

    pltpu.sync_copy(comb_hbm, comb_v)
    pltpu.sync_copy(gam_hbm, gam_v)
    pltpu.sync_copy(bet_hbm, bet_v)

    inv_d = jnp.float32(1.0 / D)

    def cid_copy(c, p):
        return pltpu.make_async_copy(cid_hbm.at[cbase + c], cid_v[p],
                                     ids_s[p])

    def cmb_copy(c, p):
        return pltpu.make_async_copy(cmb_hbm.at[cbase + c], cmb_v[p],
                                     ids_s[p])

    def ids_start(c, p):
        cid_copy(c, p).start()
        cmb_copy(c, p).start()

    def ids_wait(c, p):
        cid_copy(c, p).wait()
        cmb_copy(c, p).wait()

    def gather_copy(c, p):
        return pltpu.make_async_copy(code_hbm.at[cid_v[p]], rows_v[p], g_s[p])

    def out_copy(c, p):
        off = (cbase + c) * CHUNK
        return pltpu.make_async_copy(out_v[p], out_hbm.at[pl.ds(off, CHUNK)],
                                     o_s[p])

    gam = [gam_v[pl.ds(k * 16, 16)] for k in range(4)]
    bet = [bet_v[pl.ds(k * 16, 16)] for k in range(4)]
    iota16 = lax.iota(jnp.int32, 16)
    idx15 = jnp.full((16,), 15, jnp.int32)
    lane_idx = [jnp.full((16,), j, jnp.int32) for j in range(16)]

    zero16 = jnp.zeros((16,), jnp.float32)

    def _tree_add(parts):
        while len(parts) > 1:
            parts = [a + b for a, b in zip(parts[::2], parts[1::2])]
        return parts[0]

    def compute(c, p):
        rv, ov, iv = rows_v[p], out_v[p], cmb_v[p]

        @pl.loop(0, NGROUPS)
        def _group(g):
            cvec = iv[pl.ds(g * 16, 16)]
            s_parts = []
            q_parts = []
            for jb in range(0, 16, 4):
                toks = range(jb, jb + 4)
                a = [[rv[g * 16 + j, pl.ds(k * 16, 16)] for k in range(4)]
                     for j in toks]
                b = [[comb_v[cvec[j], pl.ds(k * 16, 16)] for k in range(4)]
                     for j in toks]
                v = [[a[i][k] + b[i][k] for k in range(4)]
                     for i in range(4)]
                for i, j in enumerate(toks):
                    for k in range(4):
                        ov[g * 16 + j, pl.ds(k * 16, 16)] = v[i][k]
                t = [(v[i][0] + v[i][1]) + (v[i][2] + v[i][3])
                     for i in range(4)]
                q = [(v[i][0] * v[i][0] + v[i][1] * v[i][1])
                     + (v[i][2] * v[i][2] + v[i][3] * v[i][3])
                     for i in range(4)]
                tc_ = [plsc.cumsum(x) for x in t]
                qc_ = [plsc.cumsum(x) for x in q]
                ts = [jnp.take_along_axis(x, idx15, axis=0) for x in tc_]
                qs = [jnp.take_along_axis(x, idx15, axis=0) for x in qc_]
                for i, j in enumerate(toks):
                    mask = iota16 == j
                    s_parts.append(jnp.where(mask, ts[i], zero16))
                    q_parts.append(jnp.where(mask, qs[i], zero16))
            s_l = _tree_add(s_parts)
            q_l = _tree_add(q_parts)
            mu_l = s_l * inv_d
            var_l = q_l * inv_d - mu_l * mu_l
            x = var_l + EPS
            i = plsc.bitcast(x, jnp.int32)
            i = jnp.int32(0x5F3759DF) - lax.shift_right_logical(i, 1)
            y = plsc.bitcast(i, jnp.float32)
            y = y * (1.5 - 0.5 * x * y * y)
            y = y * (1.5 - 0.5 * x * y * y)
            y = y * (1.5 - 0.5 * x * y * y)
            rstd_l = y
            for jb in range(0, 16, 4):
                toks = range(jb, jb + 4)
                mus = [jnp.take_along_axis(mu_l, lane_idx[j], axis=0)
                       for j in toks]
                rss = [jnp.take_along_axis(rstd_l, lane_idx[j], axis=0)
                       for j in toks]
                vk = [[ov[g * 16 + j, pl.ds(k * 16, 16)] for k in range(4)]
                      for j in toks]
                w = [[(vk[i][k] - mus[i]) * rss[i] * gam[k] + bet[k]
                      for k in range(4)] for i in range(4)]
                for i, j in enumerate(toks):
                    for k in range(4):
                        ov[g * 16 + j, pl.ds(k * 16, 16)] = w[i][k]

    ids_start(0, 0)
    ids_start(1, 1)
    ids_wait(0, 0)
    gather_copy(0, 0).start()

    @pl.loop(0, NCHUNKS // 2)
    def _chunk2(c2):
        for p in (0, 1):
            c = c2 * 2 + p
            q = 1 - p

            @pl.when(c + 1 < NCHUNKS)
            def _():
                ids_wait(c + 1, q)
                gather_copy(c + 1, q).start()

            gather_copy(c, p).wait()

            @pl.when(c >= 2)
            def _():
                out_copy(c - 2, p).wait()

            compute(c, p)

            @pl.when(c + 2 < NCHUNKS)
            def _():
                ids_start(c + 2, p)

            out_copy(c, p).start()

    out_copy(NCHUNKS - 2, 0).wait()
    out_copy(NCHUNKS - 1, 1).wait()


@jax.jit
def _ehr(cid3, cmb3, code_table, comb_table, gam, bet):
    mesh = plsc.VectorSubcoreMesh(core_axis_name="c", subcore_axis_name="s")
    return pl.kernel(
        _ehr_body,
        out_type=jax.ShapeDtypeStruct((N, D), jnp.float32),
        mesh=mesh,
        compiler_params=pltpu.CompilerParams(
            needs_layout_passes=False, use_tc_tiling_on_sc=False),
        scratch_types=[
            pltpu.VMEM((CHUNK,), jnp.int32),
            pltpu.VMEM((CHUNK,), jnp.int32),
            pltpu.VMEM((CHUNK,), jnp.int32),
            pltpu.VMEM((CHUNK,), jnp.int32),
            pltpu.VMEM((CHUNK, D), jnp.float32),
            pltpu.VMEM((CHUNK, D), jnp.float32),
            pltpu.VMEM((CHUNK, D), jnp.float32),
            pltpu.VMEM((CHUNK, D), jnp.float32),
            pltpu.VMEM((P * T, D), jnp.float32),
            pltpu.VMEM((D,), jnp.float32),
            pltpu.VMEM((D,), jnp.float32),
            pltpu.SemaphoreType.DMA,
            pltpu.SemaphoreType.DMA,
            pltpu.SemaphoreType.DMA,
            pltpu.SemaphoreType.DMA,
            pltpu.SemaphoreType.DMA,
            pltpu.SemaphoreType.DMA,
        ],
    )(cid3, cmb3, code_table, comb_table, gam, bet)


def kernel(code_ids, position_ids, segment_ids, code_table, pos_table,
           seg_table, ln_gamma, ln_beta):
    nch = N // CHUNK
    cid3 = code_ids.reshape(nch, CHUNK).astype(jnp.int32)
    cmb3 = (position_ids.astype(jnp.int32) * T
            + segment_ids.astype(jnp.int32)).reshape(nch, CHUNK)
    comb_table = (pos_table[:, None, :] + seg_table[None, :, :]).reshape(
        P * T, D)
    out = _ehr(cid3, cmb3, code_table, comb_table, ln_gamma, ln_beta)
    return out.reshape(code_ids.shape + (D,))
